# baseline (speedup 1.0000x reference)
import functools

import jax
import jax.numpy as jnp
from jax import lax
from jax.experimental import pallas as pl
from jax.experimental.pallas import tpu as pltpu
from jax.experimental.pallas import tpu_sc as plsc

_E = 64
_K = 8
_D = 1024
_DFF = 256
_T = 2048
_PAIRS = _T * _K
_B = 128
_NB = _PAIRS // _B + _E
_NPS = _NB * _B
_NC = 2
_NS = 16
_NW = _NC * _NS
_PPW = _PAIRS // _NW
_G = 64
_TPW = _T // _NW
_RB = 256


def _lane_cumsum_incl(a):
    n = a.shape[-1]
    sh = 1
    while sh < n:
        z = jnp.zeros_like(a[..., :sh])
        a = a + jnp.concatenate([z, a[..., :-sh]], axis=-1)
        sh *= 2
    return a


def _routing_body(logits_ref, bias_ref, pos8_ref, w16_ref, bemeta_ref):
    logits = logits_ref[...]
    scores = jax.nn.sigmoid(logits)
    choice = scores + bias_ref[...]

    cur = choice
    sel = jnp.zeros_like(choice)
    for _ in range(_K):
        m = jnp.max(cur, axis=1, keepdims=True)
        eq = (cur == m).astype(jnp.float32)
        ex = _lane_cumsum_incl(eq) - eq
        first = eq * (ex == 0.0).astype(jnp.float32)
        sel = sel + first
        cur = jnp.where(first > 0.0, -1e30, cur)

    w = scores * sel
    w = w / jnp.sum(w, axis=1, keepdims=True)

    ii = lax.broadcasted_iota(jnp.int32, (256, 256), 0)
    jj = lax.broadcasted_iota(jnp.int32, (256, 256), 1)
    ltri = (jj < ii).astype(jnp.bfloat16)
    acc = jnp.zeros((1, _E), jnp.float32)
    ranks = []
    for c in range(_T // 256):
        sc = sel[c * 256:(c + 1) * 256]
        r = lax.dot_general(ltri, sc.astype(jnp.bfloat16),
                            (((1,), (0,)), ((), ())),
                            preferred_element_type=jnp.float32)
        ranks.append(r + acc)
        acc = acc + jnp.sum(sc, axis=0, keepdims=True)
    rank = jnp.concatenate(ranks, axis=0)

    counts = acc
    padded = jnp.floor((counts + float(_B - 1)) / float(_B)) * float(_B)
    pcum = _lane_cumsum_incl(padded)
    pstart = pcum - padded
    pos = pstart + rank

    kidx = _lane_cumsum_incl(sel) - 1.0
    lane128 = lax.broadcasted_iota(jnp.int32, (1, 128 * _K), 1)
    pos8_cols = []
    w128 = jnp.zeros((_T, 128 * _K), jnp.float32)
    for k in range(_K):
        mk = sel * (kidx == float(k)).astype(jnp.float32)
        pos8_cols.append(jnp.sum(mk * pos, axis=1, keepdims=True))
        w8c = jnp.sum(mk * w, axis=1, keepdims=True)
        w128 = w128 + w8c * (lane128 == 128 * k).astype(jnp.float32)
    pos8_ref[...] = jnp.concatenate(pos8_cols, axis=1).astype(jnp.int32)
    w16_ref[...] = w128

    slot = lax.broadcasted_iota(jnp.int32, (1, 256), 1)
    starts = (slot * _B).astype(jnp.float32)
    pcum_col = jnp.transpose(pcum)
    be = jnp.sum((pcum_col <= starts).astype(jnp.float32), axis=0,
                 keepdims=True)
    be = jnp.minimum(be, float(_E - 1))
    nb = pcum[:, _E - 1:_E] / float(_B)
    bemeta = jnp.where(slot < _NB, be, nb)
    bemeta_ref[...] = bemeta.astype(jnp.int32)


def _mlp_body(be_ref, nb_ref, xs_ref, wgulo_ref, wguhi_ref, wd_ref, ws_ref,
              y_ref):
    b = pl.program_id(0)

    @pl.when(b < nb_ref[0])
    def _():
        xi = xs_ref[...]
        xlo = lax.bitcast_convert_type(xi << 16, jnp.float32)
        xhi = lax.bitcast_convert_type(xi & jnp.int32(-65536), jnp.float32)
        gu = lax.dot_general(xlo.astype(jnp.bfloat16), wgulo_ref[0],
                             (((1,), (1,)), ((), ())),
                             preferred_element_type=jnp.float32)
        gu = gu + lax.dot_general(xhi.astype(jnp.bfloat16), wguhi_ref[0],
                                  (((1,), (1,)), ((), ())),
                                  preferred_element_type=jnp.float32)
        h = (jax.nn.silu(gu[:, :_DFF]) * gu[:, _DFF:]).astype(jnp.bfloat16)
        wd = wd_ref[0]
        ylo = lax.dot_general(h, wd[:_D // 2], (((1,), (1,)), ((), ())),
                              preferred_element_type=jnp.float32)
        yhi = lax.dot_general(h, wd[_D // 2:], (((1,), (1,)), ((), ())),
                              preferred_element_type=jnp.float32)
        wsc = ws_ref[:, 0:1]
        lo16 = lax.shift_right_logical(
            lax.bitcast_convert_type(
                (ylo * wsc).astype(jnp.bfloat16).astype(jnp.float32),
                jnp.int32), 16)
        hi16 = lax.bitcast_convert_type(
            (yhi * wsc).astype(jnp.bfloat16).astype(jnp.float32),
            jnp.int32) & jnp.int32(-65536)
        y_ref[...] = lo16 | hi16


def _shared_body(x_ref, sgu_ref, sdn_ref, o_ref):
    gu = lax.dot_general(x_ref[...], sgu_ref[...], (((1,), (1,)), ((), ())),
                         preferred_element_type=jnp.float32)
    nsh = sgu_ref.shape[0] // 2
    h = (jax.nn.silu(gu[:, :nsh]) * gu[:, nsh:]).astype(jnp.bfloat16)
    o_ref[...] = lax.dot_general(h, sdn_ref[...], (((1,), (1,)), ((), ())),
                                 preferred_element_type=jnp.float32)


def _dispatch_body(x_hbm, tok_hbm, pos_hbm, w16_hbm, xs_hbm, ws_hbm,
                   tok_v, pos_v, rows_v, wrow_v, sem):
    c = lax.axis_index("c")
    s = lax.axis_index("s")
    base = (c * _NS + s) * _PPW

    @pl.loop(0, _PPW // _G)
    def _(i):
        off = base + i * _G
        pltpu.sync_copy(tok_hbm.at[pl.ds(off, _G)], tok_v)
        pltpu.sync_copy(pos_hbm.at[pl.ds(off, _G)], pos_v)
        pltpu.async_copy(x_hbm.at[tok_v], rows_v, sem).wait()
        pltpu.sync_copy(rows_v, xs_hbm.at[pos_v])
        pltpu.sync_copy(w16_hbm.at[pl.ds(off, _G)], wrow_v)
        pltpu.sync_copy(wrow_v, ws_hbm.at[pos_v])


def _unsort_body(y_hbm, pos_hbm, yp_hbm, pos_v, rows_v, sem):
    c = lax.axis_index("c")
    s = lax.axis_index("s")
    base = (c * _NS + s) * _PPW

    @pl.loop(0, _PPW // _G)
    def _(i):
        off = base + i * _G
        pltpu.sync_copy(pos_hbm.at[pl.ds(off, _G)], pos_v)
        pltpu.async_copy(y_hbm.at[pos_v], rows_v, sem).wait()
        pltpu.sync_copy(rows_v, yp_hbm.at[pl.ds(off, _G)])


def _reduce_body(yp_ref, sh_ref, o_ref):
    yi = yp_ref[...]
    lo = lax.bitcast_convert_type(yi << 16, jnp.float32)
    hi = lax.bitcast_convert_type(yi & jnp.int32(-65536), jnp.float32)
    lo = jnp.sum(lo.reshape(_RB, _K, _D // 2), axis=1)
    hi = jnp.sum(hi.reshape(_RB, _K, _D // 2), axis=1)
    o_ref[...] = jnp.concatenate([lo, hi], axis=1) + sh_ref[...]


def _run_dispatch(x3, tok, pos_flat, w16r):
    mesh = plsc.VectorSubcoreMesh(core_axis_name="c", subcore_axis_name="s")
    f = functools.partial(
        pl.kernel, mesh=mesh,
        out_type=(jax.ShapeDtypeStruct((_NPS, _D // 2), jnp.int32),
                  jax.ShapeDtypeStruct((_NPS, 128), jnp.float32)),
        scratch_types=[pltpu.VMEM((_G,), jnp.int32),
                       pltpu.VMEM((_G,), jnp.int32),
                       pltpu.VMEM((_G, _D // 2), jnp.int32),
                       pltpu.VMEM((_G, 128), jnp.float32),
                       pltpu.SemaphoreType.DMA],
    )(_dispatch_body)
    return f(x3, tok, pos_flat, w16r)


def _run_combine(y_sorted, pos_flat, shared_out):
    mesh = plsc.VectorSubcoreMesh(core_axis_name="c", subcore_axis_name="s")
    f = functools.partial(
        pl.kernel, mesh=mesh,
        out_type=jax.ShapeDtypeStruct((_PAIRS, _D // 2), jnp.int32),
        scratch_types=[pltpu.VMEM((_G,), jnp.int32),
                       pltpu.VMEM((_G, _D // 2), jnp.int32),
                       pltpu.SemaphoreType.DMA],
    )(_unsort_body)
    y_pairs = f(y_sorted, pos_flat)
    return pl.pallas_call(
        _reduce_body,
        grid=(_T // _RB,),
        in_specs=[
            pl.BlockSpec((_RB * _K, _D // 2), lambda t: (t, 0)),
            pl.BlockSpec((_RB, _D), lambda t: (t, 0)),
        ],
        out_specs=pl.BlockSpec((_RB, _D), lambda t: (t, 0)),
        out_shape=jax.ShapeDtypeStruct((_T, _D), jnp.float32),
    )(y_pairs, shared_out)


def kernel(hidden_states, gate_w, expert_bias, w_gate_up, w_down,
           shared_gate_up, shared_down):
    x32 = hidden_states.astype(jnp.float32)
    logits = x32 @ gate_w.astype(jnp.float32).T

    pos8, w16, bemeta = pl.pallas_call(
        _routing_body,
        in_specs=[
            pl.BlockSpec((_T, _E), lambda: (0, 0)),
            pl.BlockSpec((1, _E), lambda: (0, 0)),
        ],
        out_specs=(
            pl.BlockSpec((_T, _K), lambda: (0, 0)),
            pl.BlockSpec((_T, 128 * _K), lambda: (0, 0)),
            pl.BlockSpec((1, 256), lambda: (0, 0)),
        ),
        out_shape=(
            jax.ShapeDtypeStruct((_T, _K), jnp.int32),
            jax.ShapeDtypeStruct((_T, 128 * _K), jnp.float32),
            jax.ShapeDtypeStruct((1, 256), jnp.int32),
        ),
    )(logits, expert_bias.reshape(1, _E).astype(jnp.float32))

    pos_flat = pos8.reshape(_PAIRS)
    w16r = w16.reshape(_PAIRS, 128)
    tok = (jnp.arange(_PAIRS, dtype=jnp.int32) // _K).astype(jnp.int32)
    block_expert = bemeta.reshape(256)[:_NB]
    nblocks = bemeta.reshape(256)[_NB:_NB + 1]

    xb16 = hidden_states.astype(jnp.bfloat16)
    lo16 = lax.shift_right_logical(
        lax.bitcast_convert_type(
            xb16[:, :_D // 2].astype(jnp.float32), jnp.int32), 16)
    hi16 = lax.bitcast_convert_type(
        xb16[:, _D // 2:].astype(jnp.float32), jnp.int32) & jnp.int32(-65536)
    xi = lo16 | hi16
    x_sorted, w_sorted = _run_dispatch(xi, tok, pos_flat, w16r)

    xb = hidden_states.astype(jnp.bfloat16)
    shared_out = pl.pallas_call(
        _shared_body,
        in_specs=[
            pl.BlockSpec((_T, _D), lambda: (0, 0)),
            pl.BlockSpec(shared_gate_up.shape, lambda: (0, 0)),
            pl.BlockSpec(shared_down.shape, lambda: (0, 0)),
        ],
        out_specs=pl.BlockSpec((_T, _D), lambda: (0, 0)),
        out_shape=jax.ShapeDtypeStruct((_T, _D), jnp.float32),
    )(xb, shared_gate_up.astype(jnp.bfloat16),
      shared_down.astype(jnp.bfloat16))

    wgu16 = w_gate_up.astype(jnp.bfloat16)
    grid_spec = pltpu.PrefetchScalarGridSpec(
        num_scalar_prefetch=2,
        grid=(_NB,),
        in_specs=[
            pl.BlockSpec((_B, _D // 2), lambda b, be, nb: (b, 0)),
            pl.BlockSpec((1, 2 * _DFF, _D // 2),
                         lambda b, be, nb: (be[b], 0, 0)),
            pl.BlockSpec((1, 2 * _DFF, _D // 2),
                         lambda b, be, nb: (be[b], 0, 0)),
            pl.BlockSpec((1, _D, _DFF), lambda b, be, nb: (be[b], 0, 0)),
            pl.BlockSpec((_B, 128), lambda b, be, nb: (b, 0)),
        ],
        out_specs=pl.BlockSpec((_B, _D // 2), lambda b, be, nb: (b, 0)),
    )
    y_sorted = pl.pallas_call(
        _mlp_body,
        grid_spec=grid_spec,
        out_shape=jax.ShapeDtypeStruct((_NPS, _D // 2), jnp.int32),
        compiler_params=pltpu.CompilerParams(
            dimension_semantics=("arbitrary",)),
    )(block_expert, nblocks, x_sorted,
      wgu16[:, :, :_D // 2], wgu16[:, :, _D // 2:],
      w_down.astype(jnp.bfloat16), w_sorted)

    return _run_combine(y_sorted, pos_flat, shared_out)


# speedup vs baseline: 1.1027x; 1.1027x over previous
import functools

import jax
import jax.numpy as jnp
from jax import lax
from jax.experimental import pallas as pl
from jax.experimental.pallas import tpu as pltpu
from jax.experimental.pallas import tpu_sc as plsc

_E = 64
_K = 8
_D = 1024
_DFF = 256
_T = 2048
_PAIRS = _T * _K
_B = 128
_NB = _PAIRS // _B + _E
_NPS = _NB * _B
_NC = 2
_NS = 16
_NW = _NC * _NS
_PPW = _PAIRS // _NW
_G = 64
_TPW = _T // _NW
_RB = 256


def _lane_cumsum_incl(a):
    n = a.shape[-1]
    sh = 1
    while sh < n:
        z = jnp.zeros_like(a[..., :sh])
        a = a + jnp.concatenate([z, a[..., :-sh]], axis=-1)
        sh *= 2
    return a


def _routing_body(logits_ref, bias_ref, pos8_ref, w16_ref, bemeta_ref):
    logits = logits_ref[...]
    scores = jax.nn.sigmoid(logits)
    choice = scores + bias_ref[...]

    cur = choice
    sel = jnp.zeros_like(choice)
    for _ in range(_K):
        m = jnp.max(cur, axis=1, keepdims=True)
        eq = (cur == m).astype(jnp.float32)
        ex = _lane_cumsum_incl(eq) - eq
        first = eq * (ex == 0.0).astype(jnp.float32)
        sel = sel + first
        cur = jnp.where(first > 0.0, -1e30, cur)

    w = scores * sel
    w = w / jnp.sum(w, axis=1, keepdims=True)

    ii = lax.broadcasted_iota(jnp.int32, (256, 256), 0)
    jj = lax.broadcasted_iota(jnp.int32, (256, 256), 1)
    ltri = (jj < ii).astype(jnp.bfloat16)
    acc = jnp.zeros((1, _E), jnp.float32)
    ranks = []
    for c in range(_T // 256):
        sc = sel[c * 256:(c + 1) * 256]
        r = lax.dot_general(ltri, sc.astype(jnp.bfloat16),
                            (((1,), (0,)), ((), ())),
                            preferred_element_type=jnp.float32)
        ranks.append(r + acc)
        acc = acc + jnp.sum(sc, axis=0, keepdims=True)
    rank = jnp.concatenate(ranks, axis=0)

    counts = acc
    padded = jnp.floor((counts + float(_B - 1)) / float(_B)) * float(_B)
    pcum = _lane_cumsum_incl(padded)
    pstart = pcum - padded
    pos = pstart + rank

    kidx = _lane_cumsum_incl(sel) - 1.0
    lane128 = lax.broadcasted_iota(jnp.int32, (1, 128 * _K), 1)
    pos8_cols = []
    w128 = jnp.zeros((_T, 128 * _K), jnp.float32)
    for k in range(_K):
        mk = sel * (kidx == float(k)).astype(jnp.float32)
        pos8_cols.append(jnp.sum(mk * pos, axis=1, keepdims=True))
        w8c = jnp.sum(mk * w, axis=1, keepdims=True)
        w128 = w128 + w8c * (lane128 == 128 * k).astype(jnp.float32)
    pos8_ref[...] = jnp.concatenate(pos8_cols, axis=1).astype(jnp.int32)
    w16_ref[...] = w128

    slot = lax.broadcasted_iota(jnp.int32, (1, 256), 1)
    starts = (slot * _B).astype(jnp.float32)
    pcum_col = jnp.transpose(pcum)
    be = jnp.sum((pcum_col <= starts).astype(jnp.float32), axis=0,
                 keepdims=True)
    be = jnp.minimum(be, float(_E - 1))
    nb = pcum[:, _E - 1:_E] / float(_B)
    bemeta = jnp.where(slot < _NB, be, nb)
    bemeta_ref[...] = bemeta.astype(jnp.int32)


def _mlp_body(be_ref, nb_ref, xs_ref, wgu_ref, wd_ref, ws_ref, y_ref):
    b = pl.program_id(0)

    @pl.when(b < nb_ref[0])
    def _():
        xi = xs_ref[...]
        xlo = lax.bitcast_convert_type(xi << 16, jnp.float32)
        xhi = lax.bitcast_convert_type(xi & jnp.int32(-65536), jnp.float32)
        wgu = wgu_ref[0]
        gu = lax.dot_general(xlo.astype(jnp.bfloat16), wgu[:, :_D // 2],
                             (((1,), (1,)), ((), ())),
                             preferred_element_type=jnp.float32)
        gu = gu + lax.dot_general(xhi.astype(jnp.bfloat16), wgu[:, _D // 2:],
                                  (((1,), (1,)), ((), ())),
                                  preferred_element_type=jnp.float32)
        h = (jax.nn.silu(gu[:, :_DFF]) * gu[:, _DFF:]).astype(jnp.bfloat16)
        wd = wd_ref[0]
        ylo = lax.dot_general(h, wd[:_D // 2], (((1,), (1,)), ((), ())),
                              preferred_element_type=jnp.float32)
        yhi = lax.dot_general(h, wd[_D // 2:], (((1,), (1,)), ((), ())),
                              preferred_element_type=jnp.float32)
        wsc = ws_ref[:, 0:1]
        lo16 = lax.shift_right_logical(
            lax.bitcast_convert_type(
                (ylo * wsc).astype(jnp.bfloat16).astype(jnp.float32),
                jnp.int32), 16)
        hi16 = lax.bitcast_convert_type(
            (yhi * wsc).astype(jnp.bfloat16).astype(jnp.float32),
            jnp.int32) & jnp.int32(-65536)
        y_ref[...] = lo16 | hi16


def _shared_body(x_ref, sgu_ref, sdn_ref, o_ref):
    gu = lax.dot_general(x_ref[...], sgu_ref[...], (((1,), (1,)), ((), ())),
                         preferred_element_type=jnp.float32)
    nsh = sgu_ref.shape[0] // 2
    h = (jax.nn.silu(gu[:, :nsh]) * gu[:, nsh:]).astype(jnp.bfloat16)
    o_ref[...] = lax.dot_general(h, sdn_ref[...], (((1,), (1,)), ((), ())),
                                 preferred_element_type=jnp.float32)


def _dispatch_body(x_hbm, tok_hbm, pos_hbm, w16_hbm, xs_hbm, ws_hbm,
                   tok_v0, pos_v0, rows_v0, wrow_v0,
                   tok_v1, pos_v1, rows_v1, wrow_v1,
                   semg0, semg1, sems0, sems1):
    c = lax.axis_index("c")
    s = lax.axis_index("s")
    base = (c * _NS + s) * _PPW
    bufs = [(tok_v0, pos_v0, rows_v0, wrow_v0, semg0, sems0),
            (tok_v1, pos_v1, rows_v1, wrow_v1, semg1, sems1)]
    n = _PPW // _G
    gh = {}
    sh = {}

    def start(i):
        tok_v, pos_v, rows_v, wrow_v, semg, _ = bufs[i % 2]
        for h in sh.pop(i - 2, ()):
            h.wait()
        off = base + i * _G
        pltpu.sync_copy(tok_hbm.at[pl.ds(off, _G)], tok_v)
        pltpu.sync_copy(pos_hbm.at[pl.ds(off, _G)], pos_v)
        pltpu.sync_copy(w16_hbm.at[pl.ds(off, _G)], wrow_v)
        gh[i] = pltpu.async_copy(x_hbm.at[tok_v], rows_v, semg)

    start(0)
    for i in range(n):
        if i + 1 < n:
            start(i + 1)
        _, pos_v, rows_v, wrow_v, _, sems = bufs[i % 2]
        gh.pop(i).wait()
        sh[i] = (pltpu.async_copy(rows_v, xs_hbm.at[pos_v], sems),
                 pltpu.async_copy(wrow_v, ws_hbm.at[pos_v], sems))
    for hs in sh.values():
        for h in hs:
            h.wait()


def _unsort_body(y_hbm, pos_hbm, yp_hbm, pos_v0, rows_v0, pos_v1, rows_v1,
                 semg0, semg1, sems0, sems1):
    c = lax.axis_index("c")
    s = lax.axis_index("s")
    base = (c * _NS + s) * _PPW
    bufs = [(pos_v0, rows_v0, semg0, sems0), (pos_v1, rows_v1, semg1, sems1)]
    n = _PPW // _G
    gh = {}
    sh = {}

    def start(i):
        pos_v, rows_v, semg, _ = bufs[i % 2]
        h = sh.pop(i - 2, None)
        if h is not None:
            h.wait()
        off = base + i * _G
        pltpu.sync_copy(pos_hbm.at[pl.ds(off, _G)], pos_v)
        gh[i] = pltpu.async_copy(y_hbm.at[pos_v], rows_v, semg)

    start(0)
    for i in range(n):
        if i + 1 < n:
            start(i + 1)
        _, rows_v, _, sems = bufs[i % 2]
        gh.pop(i).wait()
        off = base + i * _G
        sh[i] = pltpu.async_copy(rows_v, yp_hbm.at[pl.ds(off, _G)], sems)
    for h in sh.values():
        h.wait()


def _reduce_body(yp_ref, sh_ref, o_ref):
    yi = yp_ref[...]
    lo = lax.bitcast_convert_type(yi << 16, jnp.float32)
    hi = lax.bitcast_convert_type(yi & jnp.int32(-65536), jnp.float32)
    lo = jnp.sum(lo.reshape(_RB, _K, _D // 2), axis=1)
    hi = jnp.sum(hi.reshape(_RB, _K, _D // 2), axis=1)
    o_ref[...] = jnp.concatenate([lo, hi], axis=1) + sh_ref[...]


def _run_dispatch(x3, tok, pos_flat, w16r):
    mesh = plsc.VectorSubcoreMesh(core_axis_name="c", subcore_axis_name="s")
    f = functools.partial(
        pl.kernel, mesh=mesh,
        out_type=(jax.ShapeDtypeStruct((_NPS, _D // 2), jnp.int32),
                  jax.ShapeDtypeStruct((_NPS, 128), jnp.float32)),
        scratch_types=[pltpu.VMEM((_G,), jnp.int32),
                       pltpu.VMEM((_G,), jnp.int32),
                       pltpu.VMEM((_G, _D // 2), jnp.int32),
                       pltpu.VMEM((_G, 128), jnp.float32),
                       pltpu.VMEM((_G,), jnp.int32),
                       pltpu.VMEM((_G,), jnp.int32),
                       pltpu.VMEM((_G, _D // 2), jnp.int32),
                       pltpu.VMEM((_G, 128), jnp.float32),
                       pltpu.SemaphoreType.DMA,
                       pltpu.SemaphoreType.DMA,
                       pltpu.SemaphoreType.DMA,
                       pltpu.SemaphoreType.DMA],
    )(_dispatch_body)
    return f(x3, tok, pos_flat, w16r)


def _run_combine(y_sorted, pos_flat, shared_out):
    mesh = plsc.VectorSubcoreMesh(core_axis_name="c", subcore_axis_name="s")
    f = functools.partial(
        pl.kernel, mesh=mesh,
        out_type=jax.ShapeDtypeStruct((_PAIRS, _D // 2), jnp.int32),
        scratch_types=[pltpu.VMEM((_G,), jnp.int32),
                       pltpu.VMEM((_G, _D // 2), jnp.int32),
                       pltpu.VMEM((_G,), jnp.int32),
                       pltpu.VMEM((_G, _D // 2), jnp.int32),
                       pltpu.SemaphoreType.DMA,
                       pltpu.SemaphoreType.DMA,
                       pltpu.SemaphoreType.DMA,
                       pltpu.SemaphoreType.DMA],
    )(_unsort_body)
    y_pairs = f(y_sorted, pos_flat)
    return pl.pallas_call(
        _reduce_body,
        grid=(_T // _RB,),
        in_specs=[
            pl.BlockSpec((_RB * _K, _D // 2), lambda t: (t, 0)),
            pl.BlockSpec((_RB, _D), lambda t: (t, 0)),
        ],
        out_specs=pl.BlockSpec((_RB, _D), lambda t: (t, 0)),
        out_shape=jax.ShapeDtypeStruct((_T, _D), jnp.float32),
    )(y_pairs, shared_out)


def kernel(hidden_states, gate_w, expert_bias, w_gate_up, w_down,
           shared_gate_up, shared_down):
    x32 = hidden_states.astype(jnp.float32)
    logits = x32 @ gate_w.astype(jnp.float32).T

    pos8, w16, bemeta = pl.pallas_call(
        _routing_body,
        in_specs=[
            pl.BlockSpec((_T, _E), lambda: (0, 0)),
            pl.BlockSpec((1, _E), lambda: (0, 0)),
        ],
        out_specs=(
            pl.BlockSpec((_T, _K), lambda: (0, 0)),
            pl.BlockSpec((_T, 128 * _K), lambda: (0, 0)),
            pl.BlockSpec((1, 256), lambda: (0, 0)),
        ),
        out_shape=(
            jax.ShapeDtypeStruct((_T, _K), jnp.int32),
            jax.ShapeDtypeStruct((_T, 128 * _K), jnp.float32),
            jax.ShapeDtypeStruct((1, 256), jnp.int32),
        ),
    )(logits, expert_bias.reshape(1, _E).astype(jnp.float32))

    pos_flat = pos8.reshape(_PAIRS)
    w16r = w16.reshape(_PAIRS, 128)
    tok = (jnp.arange(_PAIRS, dtype=jnp.int32) // _K).astype(jnp.int32)
    block_expert = bemeta.reshape(256)[:_NB]
    nblocks = bemeta.reshape(256)[_NB:_NB + 1]

    xb16 = hidden_states.astype(jnp.bfloat16)
    lo16 = lax.shift_right_logical(
        lax.bitcast_convert_type(
            xb16[:, :_D // 2].astype(jnp.float32), jnp.int32), 16)
    hi16 = lax.bitcast_convert_type(
        xb16[:, _D // 2:].astype(jnp.float32), jnp.int32) & jnp.int32(-65536)
    xi = lo16 | hi16
    x_sorted, w_sorted = _run_dispatch(xi, tok, pos_flat, w16r)

    xb = hidden_states.astype(jnp.bfloat16)
    shared_out = pl.pallas_call(
        _shared_body,
        in_specs=[
            pl.BlockSpec((_T, _D), lambda: (0, 0)),
            pl.BlockSpec(shared_gate_up.shape, lambda: (0, 0)),
            pl.BlockSpec(shared_down.shape, lambda: (0, 0)),
        ],
        out_specs=pl.BlockSpec((_T, _D), lambda: (0, 0)),
        out_shape=jax.ShapeDtypeStruct((_T, _D), jnp.float32),
    )(xb, shared_gate_up.astype(jnp.bfloat16),
      shared_down.astype(jnp.bfloat16))

    grid_spec = pltpu.PrefetchScalarGridSpec(
        num_scalar_prefetch=2,
        grid=(_NB,),
        in_specs=[
            pl.BlockSpec((_B, _D // 2), lambda b, be, nb: (b, 0)),
            pl.BlockSpec((1, 2 * _DFF, _D), lambda b, be, nb: (be[b], 0, 0)),
            pl.BlockSpec((1, _D, _DFF), lambda b, be, nb: (be[b], 0, 0)),
            pl.BlockSpec((_B, 128), lambda b, be, nb: (b, 0)),
        ],
        out_specs=pl.BlockSpec((_B, _D // 2), lambda b, be, nb: (b, 0)),
    )
    y_sorted = pl.pallas_call(
        _mlp_body,
        grid_spec=grid_spec,
        out_shape=jax.ShapeDtypeStruct((_NPS, _D // 2), jnp.int32),
        compiler_params=pltpu.CompilerParams(
            dimension_semantics=("arbitrary",)),
    )(block_expert, nblocks, x_sorted,
      w_gate_up.astype(jnp.bfloat16), w_down.astype(jnp.bfloat16), w_sorted)

    return _run_combine(y_sorted, pos_flat, shared_out)


# speedup vs baseline: 1.1174x; 1.0133x over previous
import functools

import jax
import jax.numpy as jnp
from jax import lax
from jax.experimental import pallas as pl
from jax.experimental.pallas import tpu as pltpu
from jax.experimental.pallas import tpu_sc as plsc

_E = 64
_K = 8
_D = 1024
_DFF = 256
_T = 2048
_PAIRS = _T * _K
_B = 128
_NB = _PAIRS // _B + _E
_NPS = _NB * _B
_NC = 2
_NS = 16
_NW = _NC * _NS
_PPW = _PAIRS // _NW
_G = 64
_TPW = _T // _NW
_RB = 256


def _lane_cumsum_incl(a):
    n = a.shape[-1]
    sh = 1
    while sh < n:
        z = jnp.zeros_like(a[..., :sh])
        a = a + jnp.concatenate([z, a[..., :-sh]], axis=-1)
        sh *= 2
    return a


def _routing_body(logits_ref, bias_ref, pos8_ref, w16_ref, bemeta_ref):
    logits = logits_ref[...]
    scores = jax.nn.sigmoid(logits)
    choice = scores + bias_ref[...]

    cur = choice
    sel = jnp.zeros_like(choice)
    for _ in range(_K):
        m = jnp.max(cur, axis=1, keepdims=True)
        eq = (cur == m).astype(jnp.float32)
        ex = _lane_cumsum_incl(eq) - eq
        first = eq * (ex == 0.0).astype(jnp.float32)
        sel = sel + first
        cur = jnp.where(first > 0.0, -1e30, cur)

    w = scores * sel
    w = w / jnp.sum(w, axis=1, keepdims=True)

    ii = lax.broadcasted_iota(jnp.int32, (256, 256), 0)
    jj = lax.broadcasted_iota(jnp.int32, (256, 256), 1)
    ltri = (jj < ii).astype(jnp.bfloat16)
    acc = jnp.zeros((1, _E), jnp.float32)
    ranks = []
    for c in range(_T // 256):
        sc = sel[c * 256:(c + 1) * 256]
        r = lax.dot_general(ltri, sc.astype(jnp.bfloat16),
                            (((1,), (0,)), ((), ())),
                            preferred_element_type=jnp.float32)
        ranks.append(r + acc)
        acc = acc + jnp.sum(sc, axis=0, keepdims=True)
    rank = jnp.concatenate(ranks, axis=0)

    counts = acc
    padded = jnp.floor((counts + float(_B - 1)) / float(_B)) * float(_B)
    pcum = _lane_cumsum_incl(padded)
    pstart = pcum - padded
    pos = pstart + rank

    kidx = _lane_cumsum_incl(sel) - 1.0
    lane128 = lax.broadcasted_iota(jnp.int32, (1, 128 * _K), 1)
    pos8_cols = []
    w128 = jnp.zeros((_T, 128 * _K), jnp.float32)
    for k in range(_K):
        mk = sel * (kidx == float(k)).astype(jnp.float32)
        pos8_cols.append(jnp.sum(mk * pos, axis=1, keepdims=True))
        w8c = jnp.sum(mk * w, axis=1, keepdims=True)
        w128 = w128 + w8c * (lane128 == 128 * k).astype(jnp.float32)
    pos8_ref[...] = jnp.concatenate(pos8_cols, axis=1).astype(jnp.int32)
    w16_ref[...] = w128

    slot = lax.broadcasted_iota(jnp.int32, (1, 256), 1)
    starts = (slot * _B).astype(jnp.float32)
    pcum_col = jnp.transpose(pcum)
    be = jnp.sum((pcum_col <= starts).astype(jnp.float32), axis=0,
                 keepdims=True)
    be = jnp.minimum(be, float(_E - 1))
    nb = pcum[:, _E - 1:_E] / float(_B)
    bemeta = jnp.where(slot < _NB, be, nb)
    bemeta_ref[...] = bemeta.astype(jnp.int32)


def _mlp_body(be_ref, nb_ref, xs_ref, wgu_ref, wd_ref, y_ref):
    b = pl.program_id(0)

    @pl.when(b < nb_ref[0])
    def _():
        xi = xs_ref[...]
        xlo = lax.bitcast_convert_type(xi << 16, jnp.float32)
        xhi = lax.bitcast_convert_type(xi & jnp.int32(-65536), jnp.float32)
        wgu = wgu_ref[0]
        gu = lax.dot_general(xlo.astype(jnp.bfloat16), wgu[:, :_D // 2],
                             (((1,), (1,)), ((), ())),
                             preferred_element_type=jnp.float32)
        gu = gu + lax.dot_general(xhi.astype(jnp.bfloat16), wgu[:, _D // 2:],
                                  (((1,), (1,)), ((), ())),
                                  preferred_element_type=jnp.float32)
        h = (jax.nn.silu(gu[:, :_DFF]) * gu[:, _DFF:]).astype(jnp.bfloat16)
        wd = wd_ref[0]
        ylo = lax.dot_general(h, wd[:_D // 2], (((1,), (1,)), ((), ())),
                              preferred_element_type=jnp.float32)
        yhi = lax.dot_general(h, wd[_D // 2:], (((1,), (1,)), ((), ())),
                              preferred_element_type=jnp.float32)
        lo16 = lax.shift_right_logical(
            lax.bitcast_convert_type(
                ylo.astype(jnp.bfloat16).astype(jnp.float32), jnp.int32), 16)
        hi16 = lax.bitcast_convert_type(
            yhi.astype(jnp.bfloat16).astype(jnp.float32),
            jnp.int32) & jnp.int32(-65536)
        y_ref[...] = lo16 | hi16


def _shared_body(x_ref, sgu_ref, sdn_ref, o_ref):
    gu = lax.dot_general(x_ref[...], sgu_ref[...], (((1,), (1,)), ((), ())),
                         preferred_element_type=jnp.float32)
    nsh = sgu_ref.shape[0] // 2
    h = (jax.nn.silu(gu[:, :nsh]) * gu[:, nsh:]).astype(jnp.bfloat16)
    o_ref[...] = lax.dot_general(h, sdn_ref[...], (((1,), (1,)), ((), ())),
                                 preferred_element_type=jnp.float32)


def _dispatch_body(x_hbm, tok_hbm, pos_hbm, xs_hbm,
                   tok_v0, pos_v0, rows_v0,
                   tok_v1, pos_v1, rows_v1,
                   semg0, semg1, sems0, sems1):
    c = lax.axis_index("c")
    s = lax.axis_index("s")
    base = (c * _NS + s) * _PPW
    bufs = [(tok_v0, pos_v0, rows_v0, semg0, sems0),
            (tok_v1, pos_v1, rows_v1, semg1, sems1)]
    n = _PPW // _G
    gh = {}
    sh = {}

    def start(i):
        tok_v, pos_v, rows_v, semg, _ = bufs[i % 2]
        h = sh.pop(i - 2, None)
        if h is not None:
            h.wait()
        off = base + i * _G
        pltpu.sync_copy(tok_hbm.at[pl.ds(off, _G)], tok_v)
        pltpu.sync_copy(pos_hbm.at[pl.ds(off, _G)], pos_v)
        gh[i] = pltpu.async_copy(x_hbm.at[tok_v], rows_v, semg)

    start(0)
    for i in range(n):
        if i + 1 < n:
            start(i + 1)
        _, pos_v, rows_v, _, sems = bufs[i % 2]
        gh.pop(i).wait()
        sh[i] = pltpu.async_copy(rows_v, xs_hbm.at[pos_v], sems)
    for h in sh.values():
        h.wait()


def _unsort_body(y_hbm, pos_hbm, yp_hbm, pos_v0, rows_v0, pos_v1, rows_v1,
                 semg0, semg1, sems0, sems1):
    c = lax.axis_index("c")
    s = lax.axis_index("s")
    base = (c * _NS + s) * _PPW
    bufs = [(pos_v0, rows_v0, semg0, sems0), (pos_v1, rows_v1, semg1, sems1)]
    n = _PPW // _G
    gh = {}
    sh = {}

    def start(i):
        pos_v, rows_v, semg, _ = bufs[i % 2]
        h = sh.pop(i - 2, None)
        if h is not None:
            h.wait()
        off = base + i * _G
        pltpu.sync_copy(pos_hbm.at[pl.ds(off, _G)], pos_v)
        gh[i] = pltpu.async_copy(y_hbm.at[pos_v], rows_v, semg)

    start(0)
    for i in range(n):
        if i + 1 < n:
            start(i + 1)
        _, rows_v, _, sems = bufs[i % 2]
        gh.pop(i).wait()
        off = base + i * _G
        sh[i] = pltpu.async_copy(rows_v, yp_hbm.at[pl.ds(off, _G)], sems)
    for h in sh.values():
        h.wait()


def _reduce_body(yp_ref, wp_ref, sh_ref, o_ref):
    yi = yp_ref[...]
    wc = wp_ref[:, 0:1]
    lo = lax.bitcast_convert_type(yi << 16, jnp.float32) * wc
    hi = lax.bitcast_convert_type(yi & jnp.int32(-65536), jnp.float32) * wc
    lo = jnp.sum(lo.reshape(_RB, _K, _D // 2), axis=1)
    hi = jnp.sum(hi.reshape(_RB, _K, _D // 2), axis=1)
    o_ref[...] = jnp.concatenate([lo, hi], axis=1) + sh_ref[...]


def _run_dispatch(x3, tok, pos_flat):
    mesh = plsc.VectorSubcoreMesh(core_axis_name="c", subcore_axis_name="s")
    f = functools.partial(
        pl.kernel, mesh=mesh,
        out_type=jax.ShapeDtypeStruct((_NPS, _D // 2), jnp.int32),
        scratch_types=[pltpu.VMEM((_G,), jnp.int32),
                       pltpu.VMEM((_G,), jnp.int32),
                       pltpu.VMEM((_G, _D // 2), jnp.int32),
                       pltpu.VMEM((_G,), jnp.int32),
                       pltpu.VMEM((_G,), jnp.int32),
                       pltpu.VMEM((_G, _D // 2), jnp.int32),
                       pltpu.SemaphoreType.DMA,
                       pltpu.SemaphoreType.DMA,
                       pltpu.SemaphoreType.DMA,
                       pltpu.SemaphoreType.DMA],
    )(_dispatch_body)
    return f(x3, tok, pos_flat)


def _run_combine(y_sorted, pos_flat, w16r, shared_out):
    mesh = plsc.VectorSubcoreMesh(core_axis_name="c", subcore_axis_name="s")
    f = functools.partial(
        pl.kernel, mesh=mesh,
        out_type=jax.ShapeDtypeStruct((_PAIRS, _D // 2), jnp.int32),
        scratch_types=[pltpu.VMEM((_G,), jnp.int32),
                       pltpu.VMEM((_G, _D // 2), jnp.int32),
                       pltpu.VMEM((_G,), jnp.int32),
                       pltpu.VMEM((_G, _D // 2), jnp.int32),
                       pltpu.SemaphoreType.DMA,
                       pltpu.SemaphoreType.DMA,
                       pltpu.SemaphoreType.DMA,
                       pltpu.SemaphoreType.DMA],
    )(_unsort_body)
    y_pairs = f(y_sorted, pos_flat)
    return pl.pallas_call(
        _reduce_body,
        grid=(_T // _RB,),
        in_specs=[
            pl.BlockSpec((_RB * _K, _D // 2), lambda t: (t, 0)),
            pl.BlockSpec((_RB * _K, 128), lambda t: (t, 0)),
            pl.BlockSpec((_RB, _D), lambda t: (t, 0)),
        ],
        out_specs=pl.BlockSpec((_RB, _D), lambda t: (t, 0)),
        out_shape=jax.ShapeDtypeStruct((_T, _D), jnp.float32),
    )(y_pairs, w16r, shared_out)


def kernel(hidden_states, gate_w, expert_bias, w_gate_up, w_down,
           shared_gate_up, shared_down):
    x32 = hidden_states.astype(jnp.float32)
    logits = x32 @ gate_w.astype(jnp.float32).T

    pos8, w16, bemeta = pl.pallas_call(
        _routing_body,
        in_specs=[
            pl.BlockSpec((_T, _E), lambda: (0, 0)),
            pl.BlockSpec((1, _E), lambda: (0, 0)),
        ],
        out_specs=(
            pl.BlockSpec((_T, _K), lambda: (0, 0)),
            pl.BlockSpec((_T, 128 * _K), lambda: (0, 0)),
            pl.BlockSpec((1, 256), lambda: (0, 0)),
        ),
        out_shape=(
            jax.ShapeDtypeStruct((_T, _K), jnp.int32),
            jax.ShapeDtypeStruct((_T, 128 * _K), jnp.float32),
            jax.ShapeDtypeStruct((1, 256), jnp.int32),
        ),
    )(logits, expert_bias.reshape(1, _E).astype(jnp.float32))

    pos_flat = pos8.reshape(_PAIRS)
    w16r = w16.reshape(_PAIRS, 128)
    tok = (jnp.arange(_PAIRS, dtype=jnp.int32) // _K).astype(jnp.int32)
    block_expert = bemeta.reshape(256)[:_NB]
    nblocks = bemeta.reshape(256)[_NB:_NB + 1]

    xb16 = hidden_states.astype(jnp.bfloat16)
    lo16 = lax.shift_right_logical(
        lax.bitcast_convert_type(
            xb16[:, :_D // 2].astype(jnp.float32), jnp.int32), 16)
    hi16 = lax.bitcast_convert_type(
        xb16[:, _D // 2:].astype(jnp.float32), jnp.int32) & jnp.int32(-65536)
    xi = lo16 | hi16
    x_sorted = _run_dispatch(xi, tok, pos_flat)

    xb = hidden_states.astype(jnp.bfloat16)
    shared_out = pl.pallas_call(
        _shared_body,
        in_specs=[
            pl.BlockSpec((_T, _D), lambda: (0, 0)),
            pl.BlockSpec(shared_gate_up.shape, lambda: (0, 0)),
            pl.BlockSpec(shared_down.shape, lambda: (0, 0)),
        ],
        out_specs=pl.BlockSpec((_T, _D), lambda: (0, 0)),
        out_shape=jax.ShapeDtypeStruct((_T, _D), jnp.float32),
    )(xb, shared_gate_up.astype(jnp.bfloat16),
      shared_down.astype(jnp.bfloat16))

    grid_spec = pltpu.PrefetchScalarGridSpec(
        num_scalar_prefetch=2,
        grid=(_NB,),
        in_specs=[
            pl.BlockSpec((_B, _D // 2), lambda b, be, nb: (b, 0)),
            pl.BlockSpec((1, 2 * _DFF, _D), lambda b, be, nb: (be[b], 0, 0)),
            pl.BlockSpec((1, _D, _DFF), lambda b, be, nb: (be[b], 0, 0)),
        ],
        out_specs=pl.BlockSpec((_B, _D // 2), lambda b, be, nb: (b, 0)),
    )
    y_sorted = pl.pallas_call(
        _mlp_body,
        grid_spec=grid_spec,
        out_shape=jax.ShapeDtypeStruct((_NPS, _D // 2), jnp.int32),
        compiler_params=pltpu.CompilerParams(
            dimension_semantics=("arbitrary",)),
    )(block_expert, nblocks, x_sorted,
      w_gate_up.astype(jnp.bfloat16), w_down.astype(jnp.bfloat16))

    return _run_combine(y_sorted, pos_flat, w16r, shared_out)


# speedup vs baseline: 1.2649x; 1.1320x over previous
import functools

import jax
import jax.numpy as jnp
from jax import lax
from jax.experimental import pallas as pl
from jax.experimental.pallas import tpu as pltpu
from jax.experimental.pallas import tpu_sc as plsc

_E = 64
_K = 8
_D = 1024
_DFF = 256
_T = 2048
_PAIRS = _T * _K
_B = 256
_NB = _PAIRS // _B + _E
_NPS = _NB * _B
_NC = 2
_NS = 16
_NW = _NC * _NS
_PPW = _PAIRS // _NW
_G = 64
_TPW = _T // _NW
_RB = 256


def _lane_cumsum_incl(a):
    n = a.shape[-1]
    sh = 1
    while sh < n:
        z = jnp.zeros_like(a[..., :sh])
        a = a + jnp.concatenate([z, a[..., :-sh]], axis=-1)
        sh *= 2
    return a


def _routing_body(logits_ref, bias_ref, pos8_ref, w16_ref, bemeta_ref):
    logits = logits_ref[...]
    scores = jax.nn.sigmoid(logits)
    choice = scores + bias_ref[...]

    cur = choice
    sel = jnp.zeros_like(choice)
    for _ in range(_K):
        m = jnp.max(cur, axis=1, keepdims=True)
        eq = (cur == m).astype(jnp.float32)
        ex = _lane_cumsum_incl(eq) - eq
        first = eq * (ex == 0.0).astype(jnp.float32)
        sel = sel + first
        cur = jnp.where(first > 0.0, -1e30, cur)

    w = scores * sel
    w = w / jnp.sum(w, axis=1, keepdims=True)

    ii = lax.broadcasted_iota(jnp.int32, (256, 256), 0)
    jj = lax.broadcasted_iota(jnp.int32, (256, 256), 1)
    ltri = (jj < ii).astype(jnp.bfloat16)
    acc = jnp.zeros((1, _E), jnp.float32)
    ranks = []
    for c in range(_T // 256):
        sc = sel[c * 256:(c + 1) * 256]
        r = lax.dot_general(ltri, sc.astype(jnp.bfloat16),
                            (((1,), (0,)), ((), ())),
                            preferred_element_type=jnp.float32)
        ranks.append(r + acc)
        acc = acc + jnp.sum(sc, axis=0, keepdims=True)
    rank = jnp.concatenate(ranks, axis=0)

    counts = acc
    padded = jnp.floor((counts + float(_B - 1)) / float(_B)) * float(_B)
    pcum = _lane_cumsum_incl(padded)
    pstart = pcum - padded
    pos = pstart + rank

    kidx = _lane_cumsum_incl(sel) - 1.0
    lane128 = lax.broadcasted_iota(jnp.int32, (1, 128 * _K), 1)
    pos8_cols = []
    w128 = jnp.zeros((_T, 128 * _K), jnp.float32)
    for k in range(_K):
        mk = sel * (kidx == float(k)).astype(jnp.float32)
        pos8_cols.append(jnp.sum(mk * pos, axis=1, keepdims=True))
        w8c = jnp.sum(mk * w, axis=1, keepdims=True)
        w128 = w128 + w8c * (lane128 == 128 * k).astype(jnp.float32)
    pos8_ref[...] = jnp.concatenate(pos8_cols, axis=1).astype(jnp.int32)
    w16_ref[...] = w128

    slot = lax.broadcasted_iota(jnp.int32, (1, 256), 1)
    starts = (slot * _B).astype(jnp.float32)
    pcum_col = jnp.transpose(pcum)
    be = jnp.sum((pcum_col <= starts).astype(jnp.float32), axis=0,
                 keepdims=True)
    be = jnp.minimum(be, float(_E - 1))
    nb = pcum[:, _E - 1:_E] / float(_B)
    bemeta = jnp.where(slot < _NB, be, nb)
    bemeta_ref[...] = bemeta.astype(jnp.int32)


def _mlp_body(be_ref, nb_ref, xs_ref, wgu_ref, wd_ref, y_ref):
    b = pl.program_id(0)

    @pl.when(b < nb_ref[0])
    def _():
        xi = xs_ref[...]
        xlo = lax.bitcast_convert_type(xi << 16, jnp.float32)
        xhi = lax.bitcast_convert_type(xi & jnp.int32(-65536), jnp.float32)
        wgu = wgu_ref[0]
        gu = lax.dot_general(xlo.astype(jnp.bfloat16), wgu[:, :_D // 2],
                             (((1,), (1,)), ((), ())),
                             preferred_element_type=jnp.float32)
        gu = gu + lax.dot_general(xhi.astype(jnp.bfloat16), wgu[:, _D // 2:],
                                  (((1,), (1,)), ((), ())),
                                  preferred_element_type=jnp.float32)
        h = (jax.nn.silu(gu[:, :_DFF]) * gu[:, _DFF:]).astype(jnp.bfloat16)
        wd = wd_ref[0]
        ylo = lax.dot_general(h, wd[:_D // 2], (((1,), (1,)), ((), ())),
                              preferred_element_type=jnp.float32)
        yhi = lax.dot_general(h, wd[_D // 2:], (((1,), (1,)), ((), ())),
                              preferred_element_type=jnp.float32)
        lo16 = lax.shift_right_logical(
            lax.bitcast_convert_type(
                ylo.astype(jnp.bfloat16).astype(jnp.float32), jnp.int32), 16)
        hi16 = lax.bitcast_convert_type(
            yhi.astype(jnp.bfloat16).astype(jnp.float32),
            jnp.int32) & jnp.int32(-65536)
        y_ref[...] = lo16 | hi16


def _shared_body(x_ref, sgu_ref, sdn_ref, o_ref):
    gu = lax.dot_general(x_ref[...], sgu_ref[...], (((1,), (1,)), ((), ())),
                         preferred_element_type=jnp.float32)
    nsh = sgu_ref.shape[0] // 2
    h = (jax.nn.silu(gu[:, :nsh]) * gu[:, nsh:]).astype(jnp.bfloat16)
    o_ref[...] = lax.dot_general(h, sdn_ref[...], (((1,), (1,)), ((), ())),
                                 preferred_element_type=jnp.float32)


def _dispatch_body(x_hbm, tok_hbm, pos_hbm, xs_hbm,
                   tok_v0, pos_v0, rows_v0,
                   tok_v1, pos_v1, rows_v1,
                   semg0, semg1, sems0, sems1):
    c = lax.axis_index("c")
    s = lax.axis_index("s")
    base = (c * _NS + s) * _PPW
    bufs = [(tok_v0, pos_v0, rows_v0, semg0, sems0),
            (tok_v1, pos_v1, rows_v1, semg1, sems1)]
    n = _PPW // _G
    gh = {}
    sh = {}

    def start(i):
        tok_v, pos_v, rows_v, semg, _ = bufs[i % 2]
        h = sh.pop(i - 2, None)
        if h is not None:
            h.wait()
        off = base + i * _G
        pltpu.sync_copy(tok_hbm.at[pl.ds(off, _G)], tok_v)
        pltpu.sync_copy(pos_hbm.at[pl.ds(off, _G)], pos_v)
        gh[i] = pltpu.async_copy(x_hbm.at[tok_v], rows_v, semg)

    start(0)
    for i in range(n):
        if i + 1 < n:
            start(i + 1)
        _, pos_v, rows_v, _, sems = bufs[i % 2]
        gh.pop(i).wait()
        sh[i] = pltpu.async_copy(rows_v, xs_hbm.at[pos_v], sems)
    for h in sh.values():
        h.wait()


def _unsort_body(y_hbm, pos_hbm, yp_hbm, pos_v0, rows_v0, pos_v1, rows_v1,
                 semg0, semg1, sems0, sems1):
    c = lax.axis_index("c")
    s = lax.axis_index("s")
    base = (c * _NS + s) * _PPW
    bufs = [(pos_v0, rows_v0, semg0, sems0), (pos_v1, rows_v1, semg1, sems1)]
    n = _PPW // _G
    gh = {}
    sh = {}

    def start(i):
        pos_v, rows_v, semg, _ = bufs[i % 2]
        h = sh.pop(i - 2, None)
        if h is not None:
            h.wait()
        off = base + i * _G
        pltpu.sync_copy(pos_hbm.at[pl.ds(off, _G)], pos_v)
        gh[i] = pltpu.async_copy(y_hbm.at[pos_v], rows_v, semg)

    start(0)
    for i in range(n):
        if i + 1 < n:
            start(i + 1)
        _, rows_v, _, sems = bufs[i % 2]
        gh.pop(i).wait()
        off = base + i * _G
        sh[i] = pltpu.async_copy(rows_v, yp_hbm.at[pl.ds(off, _G)], sems)
    for h in sh.values():
        h.wait()


def _reduce_body(yp_ref, wp_ref, sh_ref, o_ref):
    yi = yp_ref[...]
    wc = wp_ref[:, 0:1]
    lo = lax.bitcast_convert_type(yi << 16, jnp.float32) * wc
    hi = lax.bitcast_convert_type(yi & jnp.int32(-65536), jnp.float32) * wc
    lo = jnp.sum(lo.reshape(_RB, _K, _D // 2), axis=1)
    hi = jnp.sum(hi.reshape(_RB, _K, _D // 2), axis=1)
    o_ref[...] = jnp.concatenate([lo, hi], axis=1) + sh_ref[...]


def _run_dispatch(x3, tok, pos_flat):
    mesh = plsc.VectorSubcoreMesh(core_axis_name="c", subcore_axis_name="s")
    f = functools.partial(
        pl.kernel, mesh=mesh,
        out_type=jax.ShapeDtypeStruct((_NPS, _D // 2), jnp.int32),
        scratch_types=[pltpu.VMEM((_G,), jnp.int32),
                       pltpu.VMEM((_G,), jnp.int32),
                       pltpu.VMEM((_G, _D // 2), jnp.int32),
                       pltpu.VMEM((_G,), jnp.int32),
                       pltpu.VMEM((_G,), jnp.int32),
                       pltpu.VMEM((_G, _D // 2), jnp.int32),
                       pltpu.SemaphoreType.DMA,
                       pltpu.SemaphoreType.DMA,
                       pltpu.SemaphoreType.DMA,
                       pltpu.SemaphoreType.DMA],
    )(_dispatch_body)
    return f(x3, tok, pos_flat)


def _run_combine(y_sorted, pos_flat, w16r, shared_out):
    mesh = plsc.VectorSubcoreMesh(core_axis_name="c", subcore_axis_name="s")
    f = functools.partial(
        pl.kernel, mesh=mesh,
        out_type=jax.ShapeDtypeStruct((_PAIRS, _D // 2), jnp.int32),
        scratch_types=[pltpu.VMEM((_G,), jnp.int32),
                       pltpu.VMEM((_G, _D // 2), jnp.int32),
                       pltpu.VMEM((_G,), jnp.int32),
                       pltpu.VMEM((_G, _D // 2), jnp.int32),
                       pltpu.SemaphoreType.DMA,
                       pltpu.SemaphoreType.DMA,
                       pltpu.SemaphoreType.DMA,
                       pltpu.SemaphoreType.DMA],
    )(_unsort_body)
    y_pairs = f(y_sorted, pos_flat)
    return pl.pallas_call(
        _reduce_body,
        grid=(_T // _RB,),
        in_specs=[
            pl.BlockSpec((_RB * _K, _D // 2), lambda t: (t, 0)),
            pl.BlockSpec((_RB * _K, 128), lambda t: (t, 0)),
            pl.BlockSpec((_RB, _D), lambda t: (t, 0)),
        ],
        out_specs=pl.BlockSpec((_RB, _D), lambda t: (t, 0)),
        out_shape=jax.ShapeDtypeStruct((_T, _D), jnp.float32),
    )(y_pairs, w16r, shared_out)


def kernel(hidden_states, gate_w, expert_bias, w_gate_up, w_down,
           shared_gate_up, shared_down):
    x32 = hidden_states.astype(jnp.float32)
    logits = x32 @ gate_w.astype(jnp.float32).T

    pos8, w16, bemeta = pl.pallas_call(
        _routing_body,
        in_specs=[
            pl.BlockSpec((_T, _E), lambda: (0, 0)),
            pl.BlockSpec((1, _E), lambda: (0, 0)),
        ],
        out_specs=(
            pl.BlockSpec((_T, _K), lambda: (0, 0)),
            pl.BlockSpec((_T, 128 * _K), lambda: (0, 0)),
            pl.BlockSpec((1, 256), lambda: (0, 0)),
        ),
        out_shape=(
            jax.ShapeDtypeStruct((_T, _K), jnp.int32),
            jax.ShapeDtypeStruct((_T, 128 * _K), jnp.float32),
            jax.ShapeDtypeStruct((1, 256), jnp.int32),
        ),
    )(logits, expert_bias.reshape(1, _E).astype(jnp.float32))

    pos_flat = pos8.reshape(_PAIRS)
    w16r = w16.reshape(_PAIRS, 128)
    tok = (jnp.arange(_PAIRS, dtype=jnp.int32) // _K).astype(jnp.int32)
    block_expert = bemeta.reshape(256)[:_NB]
    nblocks = bemeta.reshape(256)[_NB:_NB + 1]

    xb16 = hidden_states.astype(jnp.bfloat16)
    lo16 = lax.shift_right_logical(
        lax.bitcast_convert_type(
            xb16[:, :_D // 2].astype(jnp.float32), jnp.int32), 16)
    hi16 = lax.bitcast_convert_type(
        xb16[:, _D // 2:].astype(jnp.float32), jnp.int32) & jnp.int32(-65536)
    xi = lo16 | hi16
    x_sorted = _run_dispatch(xi, tok, pos_flat)

    xb = hidden_states.astype(jnp.bfloat16)
    shared_out = pl.pallas_call(
        _shared_body,
        in_specs=[
            pl.BlockSpec((_T, _D), lambda: (0, 0)),
            pl.BlockSpec(shared_gate_up.shape, lambda: (0, 0)),
            pl.BlockSpec(shared_down.shape, lambda: (0, 0)),
        ],
        out_specs=pl.BlockSpec((_T, _D), lambda: (0, 0)),
        out_shape=jax.ShapeDtypeStruct((_T, _D), jnp.float32),
    )(xb, shared_gate_up.astype(jnp.bfloat16),
      shared_down.astype(jnp.bfloat16))

    grid_spec = pltpu.PrefetchScalarGridSpec(
        num_scalar_prefetch=2,
        grid=(_NB,),
        in_specs=[
            pl.BlockSpec((_B, _D // 2), lambda b, be, nb: (b, 0)),
            pl.BlockSpec((1, 2 * _DFF, _D), lambda b, be, nb: (be[b], 0, 0)),
            pl.BlockSpec((1, _D, _DFF), lambda b, be, nb: (be[b], 0, 0)),
        ],
        out_specs=pl.BlockSpec((_B, _D // 2), lambda b, be, nb: (b, 0)),
    )
    y_sorted = pl.pallas_call(
        _mlp_body,
        grid_spec=grid_spec,
        out_shape=jax.ShapeDtypeStruct((_NPS, _D // 2), jnp.int32),
        compiler_params=pltpu.CompilerParams(
            dimension_semantics=("arbitrary",)),
    )(block_expert, nblocks, x_sorted,
      w_gate_up.astype(jnp.bfloat16), w_down.astype(jnp.bfloat16))

    return _run_combine(y_sorted, pos_flat, w16r, shared_out)
